# Initial kernel scaffold; baseline (speedup 1.0000x reference)
import jax
import jax.numpy as jnp
from jax.experimental import pallas as pl


def kernel(stages, table):
    raise NotImplementedError("write your pallas kernel here")


# speedup vs baseline: 1.1801x; 1.1801x over previous
import functools

import jax
import jax.numpy as jnp
from jax import lax
from jax.experimental import pallas as pl
from jax.experimental.pallas import tpu as pltpu
from jax.experimental.pallas import tpu_sc as plsc

_NUM_STAGES = 10
_EMBED_DIM = 128
_BATCH = 16384
_NC = 2
_NS = 16
_L = 16
_NW = _NC * _NS
_BPW = _BATCH // _NW
_CHUNKS = _BPW // _L
_OH_WORDS = _BPW * _NUM_STAGES

_mesh = plsc.VectorSubcoreMesh(core_axis_name="c", subcore_axis_name="s")


@functools.partial(
    pl.kernel,
    mesh=_mesh,
    out_type=[
        jax.ShapeDtypeStruct((_BATCH, _EMBED_DIM), jnp.float32),
        jax.ShapeDtypeStruct((_BATCH * _NUM_STAGES,), jnp.float32),
    ],
    scratch_types=[
        pltpu.VMEM((_BPW,), jnp.int32),
        pltpu.VMEM((_BPW, _EMBED_DIM), jnp.float32),
        pltpu.VMEM((_OH_WORDS,), jnp.float32),
        pltpu.SemaphoreType.DMA,
    ],
    compiler_params=pltpu.CompilerParams(needs_layout_passes=False),
)
def _stage_embed_kernel(stages_hbm, table_hbm, out_emb_hbm, out_oh_hbm,
                        idx_v, rows_v, oh_v, sem):
    wid = lax.axis_index("s") * _NC + lax.axis_index("c")
    base = wid * _BPW

    pltpu.sync_copy(stages_hbm.at[pl.ds(base, _BPW)], idx_v)
    gather = pltpu.async_copy(table_hbm.at[idx_v], rows_v, sem)

    zeros = jnp.zeros((_L,), jnp.float32)

    def zero_body(i, carry):
        oh_v[pl.ds(i * _L, _L)] = zeros
        return carry

    lax.fori_loop(0, _OH_WORDS // _L, zero_body, 0)

    lane = lax.iota(jnp.int32, _L)
    ones = jnp.ones((_L,), jnp.float32)

    def oh_body(c, carry):
        s = idx_v[pl.ds(c * _L, _L)]
        pos = c * (_L * _NUM_STAGES) + lane * _NUM_STAGES + s
        plsc.store_scatter(oh_v, [pos], ones)
        return carry

    lax.fori_loop(0, _CHUNKS, oh_body, 0)

    pltpu.sync_copy(oh_v, out_oh_hbm.at[pl.ds(base * _NUM_STAGES, _OH_WORDS)])
    gather.wait()
    pltpu.sync_copy(rows_v, out_emb_hbm.at[pl.ds(base, _BPW)])


def kernel(stages, table):
    stages_i32 = stages.reshape(-1).astype(jnp.int32)
    emb, oh_flat = _stage_embed_kernel(stages_i32, table)
    return emb, oh_flat.reshape(_BATCH, _NUM_STAGES)


# speedup vs baseline: 2.3109x; 1.9582x over previous
import functools

import jax
import jax.numpy as jnp
from jax import lax
from jax.experimental import pallas as pl
from jax.experimental.pallas import tpu as pltpu
from jax.experimental.pallas import tpu_sc as plsc

_NUM_STAGES = 10
_EMBED_DIM = 128
_BATCH = 16384
_NC = 2
_NS = 16
_L = 16
_NW = _NC * _NS
_BPW = _BATCH // _NW
_CHUNKS = _BPW // _L
_VPR = _EMBED_DIM // _L
_OH_WORDS = _BPW * _NUM_STAGES
_ROW_WORDS = _BPW * _EMBED_DIM
_QUARTERS = 4
_CPQ = _CHUNKS // _QUARTERS
_QWORDS = _ROW_WORDS // _QUARTERS

_mesh = plsc.VectorSubcoreMesh(core_axis_name="c", subcore_axis_name="s")


@functools.partial(
    pl.kernel,
    mesh=_mesh,
    out_type=[
        jax.ShapeDtypeStruct((_BATCH * _EMBED_DIM,), jnp.float32),
        jax.ShapeDtypeStruct((_BATCH * _NUM_STAGES,), jnp.float32),
    ],
    scratch_types=[
        pltpu.VMEM((_BPW,), jnp.int32),
        pltpu.VMEM((_NUM_STAGES * _EMBED_DIM,), jnp.float32),
        pltpu.VMEM((_ROW_WORDS,), jnp.float32),
        pltpu.VMEM((_OH_WORDS,), jnp.float32),
        pltpu.SemaphoreType.DMA,
    ],
    compiler_params=pltpu.CompilerParams(needs_layout_passes=False),
)
def _stage_embed_kernel(stages_hbm, table_hbm, out_emb_hbm, out_oh_hbm,
                        idx_v, table_v, rows_v, oh_v, sem):
    wid = lax.axis_index("s") * _NC + lax.axis_index("c")
    base = wid * _BPW

    pltpu.sync_copy(stages_hbm.at[pl.ds(base, _BPW)], idx_v)
    pltpu.sync_copy(table_hbm, table_v)

    lane = lax.iota(jnp.int32, _L)
    ones = jnp.ones((_L,), jnp.float32)
    zeros = jnp.zeros((_L,), jnp.float32)

    def chunk_body(c, carry):
        s_chunk = idx_v[pl.ds(c * _L, _L)]
        for k in range(_L):
            src = s_chunk[k] * _EMBED_DIM
            dst = (c * _L + k) * _EMBED_DIM
            for v in range(_VPR):
                rows_v[pl.ds(dst + v * _L, _L)] = table_v[pl.ds(src + v * _L, _L)]
        oh_base = c * (_L * _NUM_STAGES)
        for v in range(_NUM_STAGES):
            oh_v[pl.ds(oh_base + v * _L, _L)] = zeros
        pos = oh_base + lane * _NUM_STAGES + s_chunk
        plsc.store_scatter(oh_v, [pos], ones)
        return carry

    copies = []
    for q in range(_QUARTERS):
        lax.fori_loop(q * _CPQ, (q + 1) * _CPQ, chunk_body, 0)
        copies.append(
            pltpu.async_copy(
                rows_v.at[pl.ds(q * _QWORDS, _QWORDS)],
                out_emb_hbm.at[pl.ds(base * _EMBED_DIM + q * _QWORDS, _QWORDS)],
                sem,
            )
        )
    copies.append(
        pltpu.async_copy(
            oh_v, out_oh_hbm.at[pl.ds(base * _NUM_STAGES, _OH_WORDS)], sem
        )
    )
    for cp in copies:
        cp.wait()


def kernel(stages, table):
    stages_i32 = stages.reshape(-1).astype(jnp.int32)
    emb_flat, oh_flat = _stage_embed_kernel(stages_i32, table.reshape(-1))
    return (
        emb_flat.reshape(_BATCH, _EMBED_DIM),
        oh_flat.reshape(_BATCH, _NUM_STAGES),
    )


# speedup vs baseline: 2.7957x; 1.2098x over previous
import functools

import jax
import jax.numpy as jnp
from jax import lax
from jax.experimental import pallas as pl
from jax.experimental.pallas import tpu as pltpu
from jax.experimental.pallas import tpu_sc as plsc

_NUM_STAGES = 10
_EMBED_DIM = 128
_BATCH = 16384
_NC = 2
_NS = 16
_L = 16
_NW = _NC * _NS
_BPW = _BATCH // _NW
_CHUNKS = _BPW // _L
_VPR = _EMBED_DIM // _L
_OH_WORDS = _BPW * _NUM_STAGES
_ROW_WORDS = _BPW * _EMBED_DIM
_QUARTERS = 4
_CPQ = _CHUNKS // _QUARTERS
_QWORDS = _ROW_WORDS // _QUARTERS

_mesh = plsc.VectorSubcoreMesh(core_axis_name="c", subcore_axis_name="s")


@functools.partial(
    pl.kernel,
    mesh=_mesh,
    out_type=[
        jax.ShapeDtypeStruct((_BATCH * _EMBED_DIM,), jnp.float32),
        jax.ShapeDtypeStruct((_BATCH * _NUM_STAGES,), jnp.float32),
    ],
    scratch_types=[
        pltpu.VMEM((_BPW,), jnp.int32),
        pltpu.VMEM((_NUM_STAGES * _EMBED_DIM,), jnp.float32),
        pltpu.VMEM((_ROW_WORDS,), jnp.float32),
        pltpu.VMEM((_OH_WORDS,), jnp.float32),
        pltpu.SemaphoreType.DMA,
    ],
    compiler_params=pltpu.CompilerParams(needs_layout_passes=False),
)
def _stage_embed_kernel(stages_hbm, table_hbm, out_emb_hbm, out_oh_hbm,
                        idx_v, table_v, rows_v, oh_v, sem):
    wid = lax.axis_index("s") * _NC + lax.axis_index("c")
    base = wid * _BPW

    pltpu.sync_copy(stages_hbm.at[pl.ds(base, _BPW)], idx_v)
    pltpu.sync_copy(table_hbm, table_v)

    lane = lax.iota(jnp.int32, _L)
    ones = jnp.ones((_L,), jnp.float32)
    zeros = jnp.zeros((_L,), jnp.float32)

    def chunk_body(c, carry):
        s_chunk = idx_v[pl.ds(c * _L, _L)]
        oh_base = c * (_L * _NUM_STAGES)
        for v in range(_NUM_STAGES):
            oh_v[pl.ds(oh_base + v * _L, _L)] = zeros
        pos = oh_base + lane * _NUM_STAGES + s_chunk
        plsc.store_scatter(oh_v, [pos], ones)
        srcs = [s_chunk[k] * _EMBED_DIM for k in range(_L)]
        dst0 = c * _L * _EMBED_DIM
        for k in range(0, _L, 2):
            vals = [
                table_v[pl.ds(srcs[k + half] + v * _L, _L)]
                for half in range(2)
                for v in range(_VPR)
            ]
            for half in range(2):
                dst = dst0 + (k + half) * _EMBED_DIM
                for v in range(_VPR):
                    rows_v[pl.ds(dst + v * _L, _L)] = vals[half * _VPR + v]
        return carry

    copies = []
    for q in range(_QUARTERS):
        lax.fori_loop(q * _CPQ, (q + 1) * _CPQ, chunk_body, 0)
        copies.append(
            pltpu.async_copy(
                rows_v.at[pl.ds(q * _QWORDS, _QWORDS)],
                out_emb_hbm.at[pl.ds(base * _EMBED_DIM + q * _QWORDS, _QWORDS)],
                sem,
            )
        )
    copies.append(
        pltpu.async_copy(
            oh_v, out_oh_hbm.at[pl.ds(base * _NUM_STAGES, _OH_WORDS)], sem
        )
    )
    for cp in copies:
        cp.wait()


def kernel(stages, table):
    stages_i32 = stages.reshape(-1).astype(jnp.int32)
    emb_flat, oh_flat = _stage_embed_kernel(stages_i32, table.reshape(-1))
    return (
        emb_flat.reshape(_BATCH, _EMBED_DIM),
        oh_flat.reshape(_BATCH, _NUM_STAGES),
    )


# speedup vs baseline: 2.7991x; 1.0012x over previous
import functools

import jax
import jax.numpy as jnp
from jax import lax
from jax.experimental import pallas as pl
from jax.experimental.pallas import tpu as pltpu
from jax.experimental.pallas import tpu_sc as plsc

_NUM_STAGES = 10
_EMBED_DIM = 128
_BATCH = 16384
_NC = 2
_NS = 16
_L = 16
_NW = _NC * _NS
_BPW = _BATCH // _NW
_CHUNKS = _BPW // _L
_VPR = _EMBED_DIM // _L
_OH_WORDS = _BPW * _NUM_STAGES
_QUARTERS = 4
_CPQ = _CHUNKS // _QUARTERS
_RPQ = _BPW // _QUARTERS

_mesh = plsc.VectorSubcoreMesh(core_axis_name="c", subcore_axis_name="s")


@functools.partial(
    pl.kernel,
    mesh=_mesh,
    out_type=[
        jax.ShapeDtypeStruct((_BATCH, _EMBED_DIM), jnp.float32),
        jax.ShapeDtypeStruct((_BATCH * _NUM_STAGES,), jnp.float32),
    ],
    scratch_types=[
        pltpu.VMEM((_BPW,), jnp.int32),
        pltpu.VMEM((_NUM_STAGES * _EMBED_DIM,), jnp.float32),
        pltpu.VMEM((_BPW, _EMBED_DIM), jnp.float32),
        pltpu.VMEM((_OH_WORDS,), jnp.float32),
        pltpu.SemaphoreType.DMA,
    ],
    compiler_params=pltpu.CompilerParams(needs_layout_passes=False),
)
def _stage_embed_kernel(stages_hbm, table_hbm, out_emb_hbm, out_oh_hbm,
                        idx_v, table_v, rows_v, oh_v, sem):
    wid = lax.axis_index("s") * _NC + lax.axis_index("c")
    base = wid * _BPW

    pltpu.sync_copy(stages_hbm.at[pl.ds(base, _BPW)], idx_v)
    pltpu.sync_copy(table_hbm, table_v)

    lane = lax.iota(jnp.int32, _L)
    ones = jnp.ones((_L,), jnp.float32)
    zeros = jnp.zeros((_L,), jnp.float32)

    def chunk_body(c, carry):
        s_chunk = idx_v[pl.ds(c * _L, _L)]
        oh_base = c * (_L * _NUM_STAGES)
        for v in range(_NUM_STAGES):
            oh_v[pl.ds(oh_base + v * _L, _L)] = zeros
        pos = oh_base + lane * _NUM_STAGES + s_chunk
        plsc.store_scatter(oh_v, [pos], ones)
        srcs = [s_chunk[k] * _EMBED_DIM for k in range(_L)]
        for k in range(0, _L, 2):
            vals = [
                table_v[pl.ds(srcs[k + half] + v * _L, _L)]
                for half in range(2)
                for v in range(_VPR)
            ]
            for half in range(2):
                row = c * _L + k + half
                for v in range(_VPR):
                    rows_v[row, pl.ds(v * _L, _L)] = vals[half * _VPR + v]
        return carry

    copies = []
    for q in range(_QUARTERS):
        lax.fori_loop(q * _CPQ, (q + 1) * _CPQ, chunk_body, 0)
        copies.append(
            pltpu.async_copy(
                rows_v.at[pl.ds(q * _RPQ, _RPQ)],
                out_emb_hbm.at[pl.ds(base + q * _RPQ, _RPQ)],
                sem,
            )
        )
    copies.append(
        pltpu.async_copy(
            oh_v, out_oh_hbm.at[pl.ds(base * _NUM_STAGES, _OH_WORDS)], sem
        )
    )
    for cp in copies:
        cp.wait()


def kernel(stages, table):
    stages_i32 = stages.reshape(-1).astype(jnp.int32)
    emb, oh_flat = _stage_embed_kernel(stages_i32, table.reshape(-1))
    return emb, oh_flat.reshape(_BATCH, _NUM_STAGES)


# speedup vs baseline: 4.3923x; 1.5692x over previous
import functools

import jax
import jax.numpy as jnp
from jax import lax
from jax.experimental import pallas as pl
from jax.experimental.pallas import tpu as pltpu
from jax.experimental.pallas import tpu_sc as plsc

_NUM_STAGES = 10
_EMBED_DIM = 128
_BATCH = 16384
_NC = 2
_NS = 16
_L = 16
_NW = _NC * _NS
_BPW = _BATCH // _NW
_CHUNKS = _BPW // _L
_VPR = _EMBED_DIM // _L
_OH_WORDS = _BPW * _NUM_STAGES
_QUARTERS = 4
_CPQ = _CHUNKS // _QUARTERS
_RPQ = _BPW // _QUARTERS

_mesh = plsc.VectorSubcoreMesh(core_axis_name="c", subcore_axis_name="s")


@functools.partial(
    pl.kernel,
    mesh=_mesh,
    out_type=[
        jax.ShapeDtypeStruct((_BATCH, _EMBED_DIM), jnp.float32),
        jax.ShapeDtypeStruct((_NUM_STAGES, _BATCH), jnp.float32),
    ],
    scratch_types=[
        pltpu.VMEM((_BPW,), jnp.int32),
        pltpu.VMEM((_NUM_STAGES * _EMBED_DIM,), jnp.float32),
        pltpu.VMEM((_BPW, _EMBED_DIM), jnp.float32),
        pltpu.VMEM((_NUM_STAGES, _BPW), jnp.float32),
        pltpu.SemaphoreType.DMA,
    ],
    compiler_params=pltpu.CompilerParams(needs_layout_passes=False),
)
def _stage_embed_kernel(stages_hbm, table_hbm, out_emb_hbm, out_oh_hbm,
                        idx_v, table_v, rows_v, oh_v, sem):
    wid = lax.axis_index("s") * _NC + lax.axis_index("c")
    base = wid * _BPW

    pltpu.sync_copy(stages_hbm.at[pl.ds(base, _BPW)], idx_v)
    pltpu.sync_copy(table_hbm, table_v)

    ones = jnp.ones((_L,), jnp.float32)
    zeros = jnp.zeros((_L,), jnp.float32)

    def chunk_body(c, carry):
        s_chunk = idx_v[pl.ds(c * _L, _L)]
        for j in range(_NUM_STAGES):
            oh_v[j, pl.ds(c * _L, _L)] = jnp.where(s_chunk == j, ones, zeros)
        srcs = [s_chunk[k] * _EMBED_DIM for k in range(_L)]
        for k in range(0, _L, 2):
            vals = [
                table_v[pl.ds(srcs[k + half] + v * _L, _L)]
                for half in range(2)
                for v in range(_VPR)
            ]
            for half in range(2):
                row = c * _L + k + half
                for v in range(_VPR):
                    rows_v[row, pl.ds(v * _L, _L)] = vals[half * _VPR + v]
        return carry

    copies = []
    for q in range(_QUARTERS):
        lax.fori_loop(q * _CPQ, (q + 1) * _CPQ, chunk_body, 0)
        copies.append(
            pltpu.async_copy(
                rows_v.at[pl.ds(q * _RPQ, _RPQ)],
                out_emb_hbm.at[pl.ds(base + q * _RPQ, _RPQ)],
                sem,
            )
        )
    copies.append(
        pltpu.async_copy(
            oh_v, out_oh_hbm.at[:, pl.ds(base, _BPW)], sem
        )
    )
    for cp in copies:
        cp.wait()


def kernel(stages, table):
    stages_i32 = stages.reshape(-1).astype(jnp.int32)
    emb, oh_t = _stage_embed_kernel(stages_i32, table.reshape(-1))
    return emb, oh_t.T


# speedup vs baseline: 4.5476x; 1.0353x over previous
import functools

import jax
import jax.numpy as jnp
from jax import lax
from jax.experimental import pallas as pl
from jax.experimental.pallas import tpu as pltpu
from jax.experimental.pallas import tpu_sc as plsc

_NUM_STAGES = 10
_EMBED_DIM = 128
_BATCH = 16384
_NC = 2
_NS = 16
_L = 16
_NW = _NC * _NS
_BPW = _BATCH // _NW
_CHUNKS = _BPW // _L
_VPR = _EMBED_DIM // _L
_OH_WORDS = _BPW * _NUM_STAGES
_QUARTERS = 4
_CPQ = _CHUNKS // _QUARTERS
_RPQ = _BPW // _QUARTERS

_mesh = plsc.VectorSubcoreMesh(core_axis_name="c", subcore_axis_name="s")


@functools.partial(
    pl.kernel,
    mesh=_mesh,
    out_type=[
        jax.ShapeDtypeStruct((_BATCH, _EMBED_DIM), jnp.float32),
        jax.ShapeDtypeStruct((_NUM_STAGES, _BATCH), jnp.float32),
    ],
    scratch_types=[
        pltpu.VMEM((_BPW,), jnp.int32),
        pltpu.VMEM((_NUM_STAGES * _EMBED_DIM,), jnp.float32),
        pltpu.VMEM((_BPW, _EMBED_DIM), jnp.float32),
        pltpu.VMEM((_NUM_STAGES, _BPW), jnp.float32),
        pltpu.SemaphoreType.DMA,
    ],
    compiler_params=pltpu.CompilerParams(needs_layout_passes=False),
)
def _stage_embed_kernel(stages_hbm, table_hbm, out_emb_hbm, out_oh_hbm,
                        idx_v, table_v, rows_v, oh_v, sem):
    wid = lax.axis_index("s") * _NC + lax.axis_index("c")
    base = wid * _BPW

    pltpu.sync_copy(stages_hbm.at[pl.ds(base, _BPW)], idx_v)
    pltpu.sync_copy(table_hbm, table_v)

    ones = jnp.ones((_L,), jnp.float32)
    zeros = jnp.zeros((_L,), jnp.float32)

    def chunk_body(c, carry):
        s_chunk = idx_v[pl.ds(c * _L, _L)]
        for j in range(_NUM_STAGES):
            oh_v[j, pl.ds(c * _L, _L)] = jnp.where(s_chunk == j, ones, zeros)
        srcs = [s_chunk[k] * _EMBED_DIM for k in range(_L)]
        prev = None
        for k in range(_L + 1):
            cur = []
            for v in range(_VPR):
                if k < _L:
                    cur.append(table_v[pl.ds(srcs[k] + v * _L, _L)])
                if prev is not None:
                    rows_v[c * _L + k - 1, pl.ds(v * _L, _L)] = prev[v]
            prev = cur
        return carry

    copies = []
    for q in range(_QUARTERS):
        lax.fori_loop(q * _CPQ, (q + 1) * _CPQ, chunk_body, 0)
        copies.append(
            pltpu.async_copy(
                rows_v.at[pl.ds(q * _RPQ, _RPQ)],
                out_emb_hbm.at[pl.ds(base + q * _RPQ, _RPQ)],
                sem,
            )
        )
    copies.append(
        pltpu.async_copy(
            oh_v, out_oh_hbm.at[:, pl.ds(base, _BPW)], sem
        )
    )
    for cp in copies:
        cp.wait()


def kernel(stages, table):
    stages_i32 = stages.reshape(-1).astype(jnp.int32)
    emb, oh_t = _stage_embed_kernel(stages_i32, table.reshape(-1))
    return emb, oh_t.T


# speedup vs baseline: 4.5636x; 1.0035x over previous
import functools

import jax
import jax.numpy as jnp
from jax import lax
from jax.experimental import pallas as pl
from jax.experimental.pallas import tpu as pltpu
from jax.experimental.pallas import tpu_sc as plsc

_NUM_STAGES = 10
_EMBED_DIM = 128
_BATCH = 16384
_NC = 2
_NS = 16
_L = 16
_NW = _NC * _NS
_BPW = _BATCH // _NW
_CHUNKS = _BPW // _L
_VPR = _EMBED_DIM // _L
_OH_WORDS = _BPW * _NUM_STAGES
_QUARTERS = 4
_CPQ = _CHUNKS // _QUARTERS
_RPQ = _BPW // _QUARTERS

_mesh = plsc.VectorSubcoreMesh(core_axis_name="c", subcore_axis_name="s")


@functools.partial(
    pl.kernel,
    mesh=_mesh,
    out_type=[
        jax.ShapeDtypeStruct((_BATCH, _EMBED_DIM), jnp.float32),
        jax.ShapeDtypeStruct((_NUM_STAGES, _BATCH), jnp.float32),
    ],
    scratch_types=[
        pltpu.VMEM((_BPW,), jnp.int32),
        pltpu.VMEM((_NUM_STAGES * _EMBED_DIM,), jnp.float32),
        pltpu.VMEM((_BPW, _EMBED_DIM), jnp.float32),
        pltpu.VMEM((_NUM_STAGES, _BPW), jnp.float32),
        pltpu.SemaphoreType.DMA,
    ],
    compiler_params=pltpu.CompilerParams(
        needs_layout_passes=False, skip_device_barrier=True
    ),
)
def _stage_embed_kernel(stages_hbm, table_hbm, out_emb_hbm, out_oh_hbm,
                        idx_v, table_v, rows_v, oh_v, sem):
    wid = lax.axis_index("s") * _NC + lax.axis_index("c")
    base = wid * _BPW

    pltpu.sync_copy(stages_hbm.at[pl.ds(base, _BPW)], idx_v)
    pltpu.sync_copy(table_hbm, table_v)

    ones = jnp.ones((_L,), jnp.float32)
    zeros = jnp.zeros((_L,), jnp.float32)

    def chunk_body(c, carry):
        s_chunk = idx_v[pl.ds(c * _L, _L)]
        for j in range(_NUM_STAGES):
            oh_v[j, pl.ds(c * _L, _L)] = jnp.where(s_chunk == j, ones, zeros)
        srcs = [s_chunk[k] * _EMBED_DIM for k in range(_L)]
        prev = None
        for k in range(_L + 1):
            cur = []
            for v in range(_VPR):
                if k < _L:
                    cur.append(table_v[pl.ds(srcs[k] + v * _L, _L)])
                if prev is not None:
                    rows_v[c * _L + k - 1, pl.ds(v * _L, _L)] = prev[v]
            prev = cur
        return carry

    copies = []
    for q in range(_QUARTERS):
        lax.fori_loop(q * _CPQ, (q + 1) * _CPQ, chunk_body, 0)
        copies.append(
            pltpu.async_copy(
                rows_v.at[pl.ds(q * _RPQ, _RPQ)],
                out_emb_hbm.at[pl.ds(base + q * _RPQ, _RPQ)],
                sem,
            )
        )
    copies.append(
        pltpu.async_copy(
            oh_v, out_oh_hbm.at[:, pl.ds(base, _BPW)], sem
        )
    )
    for cp in copies:
        cp.wait()


def kernel(stages, table):
    stages_i32 = stages.reshape(-1).astype(jnp.int32)
    emb, oh_t = _stage_embed_kernel(stages_i32, table.reshape(-1))
    return emb, oh_t.T


# speedup vs baseline: 4.7347x; 1.0375x over previous
import functools

import jax
import jax.numpy as jnp
from jax import lax
from jax.experimental import pallas as pl
from jax.experimental.pallas import tpu as pltpu
from jax.experimental.pallas import tpu_sc as plsc

_NUM_STAGES = 10
_EMBED_DIM = 128
_BATCH = 16384
_NC = 2
_NS = 16
_L = 16
_NW = _NC * _NS
_BPW = _BATCH // _NW
_CHUNKS = _BPW // _L
_VPR = _EMBED_DIM // _L
_OH_WORDS = _BPW * _NUM_STAGES
_QUARTERS = 4
_CPQ = _CHUNKS // _QUARTERS
_RPQ = _BPW // _QUARTERS

_mesh = plsc.VectorSubcoreMesh(core_axis_name="c", subcore_axis_name="s")


@functools.partial(
    pl.kernel,
    mesh=_mesh,
    out_type=[
        jax.ShapeDtypeStruct((_BATCH, _EMBED_DIM), jnp.float32),
        jax.ShapeDtypeStruct((_NUM_STAGES, _BATCH), jnp.float32),
    ],
    scratch_types=[
        pltpu.VMEM((_BPW,), jnp.int32),
        pltpu.VMEM((_NUM_STAGES * _EMBED_DIM,), jnp.float32),
        pltpu.VMEM((_BPW, _EMBED_DIM), jnp.float32),
        pltpu.VMEM((_NUM_STAGES, _BPW), jnp.float32),
        pltpu.SemaphoreType.DMA,
    ],
    compiler_params=pltpu.CompilerParams(
        needs_layout_passes=False, skip_device_barrier=True
    ),
)
def _stage_embed_kernel(stages_hbm, table_hbm, out_emb_hbm, out_oh_hbm,
                        idx_v, table_v, rows_v, oh_v, sem):
    wid = lax.axis_index("s") * _NC + lax.axis_index("c")
    base = wid * _BPW

    pltpu.sync_copy(stages_hbm.at[pl.ds(base, _BPW)], idx_v)
    pltpu.sync_copy(table_hbm, table_v)

    ones = jnp.ones((_L,), jnp.float32)
    zeros = jnp.zeros((_L,), jnp.float32)

    def chunk_body(c, carry):
        s_chunk = idx_v[pl.ds(c * _L, _L)]
        for j in range(_NUM_STAGES):
            oh_v[j, pl.ds(c * _L, _L)] = jnp.where(s_chunk == j, ones, zeros)
        srcs = [s_chunk[k] * _EMBED_DIM for k in range(_L)]
        prev = None
        for k in range(_L + 1):
            cur = []
            for v in range(_VPR):
                if k < _L:
                    cur.append(table_v[pl.ds(srcs[k] + v * _L, _L)])
                if prev is not None:
                    rows_v[c * _L + k - 1, pl.ds(v * _L, _L)] = prev[v]
            prev = cur

        @pl.when(c % _CPQ == _CPQ - 1)
        def _():
            q = c // _CPQ
            pltpu.make_async_copy(
                rows_v.at[pl.ds(q * _RPQ, _RPQ)],
                out_emb_hbm.at[pl.ds(base + q * _RPQ, _RPQ)],
                sem,
            ).start()

        return carry

    lax.fori_loop(0, _CHUNKS, chunk_body, 0)
    oh_copy = pltpu.async_copy(oh_v, out_oh_hbm.at[:, pl.ds(base, _BPW)], sem)
    for q in range(_QUARTERS):
        pltpu.make_async_copy(
            rows_v.at[pl.ds(q * _RPQ, _RPQ)],
            out_emb_hbm.at[pl.ds(base + q * _RPQ, _RPQ)],
            sem,
        ).wait()
    oh_copy.wait()


def kernel(stages, table):
    stages_i32 = stages.reshape(-1).astype(jnp.int32)
    emb, oh_t = _stage_embed_kernel(stages_i32, table.reshape(-1))
    return emb, oh_t.T


# speedup vs baseline: 4.8791x; 1.0305x over previous
import functools

import jax
import jax.numpy as jnp
from jax import lax
from jax.experimental import pallas as pl
from jax.experimental.pallas import tpu as pltpu
from jax.experimental.pallas import tpu_sc as plsc

_NUM_STAGES = 10
_EMBED_DIM = 128
_BATCH = 16384
_NC = 2
_NS = 16
_L = 16
_NW = _NC * _NS
_BPW = _BATCH // _NW
_CHUNKS = _BPW // _L
_VPR = _EMBED_DIM // _L
_OH_WORDS = _BPW * _NUM_STAGES
_QUARTERS = 8
_CPQ = _CHUNKS // _QUARTERS
_RPQ = _BPW // _QUARTERS

_mesh = plsc.VectorSubcoreMesh(core_axis_name="c", subcore_axis_name="s")


@functools.partial(
    pl.kernel,
    mesh=_mesh,
    out_type=[
        jax.ShapeDtypeStruct((_BATCH, _EMBED_DIM), jnp.float32),
        jax.ShapeDtypeStruct((_NUM_STAGES, _BATCH), jnp.float32),
    ],
    scratch_types=[
        pltpu.VMEM((_BPW,), jnp.int32),
        pltpu.VMEM((_NUM_STAGES * _EMBED_DIM,), jnp.float32),
        pltpu.VMEM((_BPW, _EMBED_DIM), jnp.float32),
        pltpu.VMEM((_NUM_STAGES, _BPW), jnp.float32),
        pltpu.SemaphoreType.DMA,
    ],
    compiler_params=pltpu.CompilerParams(
        needs_layout_passes=False, skip_device_barrier=True
    ),
)
def _stage_embed_kernel(stages_hbm, table_hbm, out_emb_hbm, out_oh_hbm,
                        idx_v, table_v, rows_v, oh_v, sem):
    wid = lax.axis_index("s") * _NC + lax.axis_index("c")
    base = wid * _BPW

    idx_cp = pltpu.async_copy(stages_hbm.at[pl.ds(base, _BPW)], idx_v, sem)
    tab_cp = pltpu.async_copy(table_hbm, table_v, sem)
    idx_cp.wait()
    tab_cp.wait()

    ones = jnp.ones((_L,), jnp.float32)
    zeros = jnp.zeros((_L,), jnp.float32)

    def chunk_body(c, carry):
        s_chunk = idx_v[pl.ds(c * _L, _L)]
        for j in range(_NUM_STAGES):
            oh_v[j, pl.ds(c * _L, _L)] = jnp.where(s_chunk == j, ones, zeros)
        srcs = [s_chunk[k] * _EMBED_DIM for k in range(_L)]
        prev = None
        for k in range(_L + 1):
            cur = []
            for v in range(_VPR):
                if k < _L:
                    cur.append(table_v[pl.ds(srcs[k] + v * _L, _L)])
                if prev is not None:
                    rows_v[c * _L + k - 1, pl.ds(v * _L, _L)] = prev[v]
            prev = cur

        @pl.when(c % _CPQ == _CPQ - 1)
        def _():
            q = c // _CPQ
            pltpu.make_async_copy(
                rows_v.at[pl.ds(q * _RPQ, _RPQ)],
                out_emb_hbm.at[pl.ds(base + q * _RPQ, _RPQ)],
                sem,
            ).start()

        return carry

    lax.fori_loop(0, _CHUNKS, chunk_body, 0)
    oh_copy = pltpu.async_copy(oh_v, out_oh_hbm.at[:, pl.ds(base, _BPW)], sem)
    for q in range(_QUARTERS):
        pltpu.make_async_copy(
            rows_v.at[pl.ds(q * _RPQ, _RPQ)],
            out_emb_hbm.at[pl.ds(base + q * _RPQ, _RPQ)],
            sem,
        ).wait()
    oh_copy.wait()


def kernel(stages, table):
    stages_i32 = stages.reshape(-1).astype(jnp.int32)
    emb, oh_t = _stage_embed_kernel(stages_i32, table.reshape(-1))
    return emb, oh_t.T


# speedup vs baseline: 4.9568x; 1.0159x over previous
import functools

import jax
import jax.numpy as jnp
from jax import lax
from jax.experimental import pallas as pl
from jax.experimental.pallas import tpu as pltpu
from jax.experimental.pallas import tpu_sc as plsc

_NUM_STAGES = 10
_EMBED_DIM = 128
_BATCH = 16384
_NC = 2
_NS = 16
_L = 16
_NW = _NC * _NS
_BPW = _BATCH // _NW
_CHUNKS = _BPW // _L
_VPR = _EMBED_DIM // _L
_OH_WORDS = _BPW * _NUM_STAGES
_QUARTERS = 8
_CPQ = _CHUNKS // _QUARTERS
_RPQ = _BPW // _QUARTERS

_mesh = plsc.VectorSubcoreMesh(core_axis_name="c", subcore_axis_name="s")


@functools.partial(
    pl.kernel,
    mesh=_mesh,
    out_type=[
        jax.ShapeDtypeStruct((_BATCH, _EMBED_DIM), jnp.float32),
        jax.ShapeDtypeStruct((_NUM_STAGES, _BATCH), jnp.float32),
    ],
    scratch_types=[
        pltpu.VMEM((_BPW,), jnp.int32),
        pltpu.VMEM((_NUM_STAGES * _EMBED_DIM,), jnp.float32),
        pltpu.VMEM((_BPW, _EMBED_DIM), jnp.float32),
        pltpu.VMEM((_NUM_STAGES, _BPW), jnp.float32),
        pltpu.SemaphoreType.DMA,
    ],
    compiler_params=pltpu.CompilerParams(
        needs_layout_passes=False, skip_device_barrier=True
    ),
)
def _stage_embed_kernel(stages_hbm, table_hbm, out_emb_hbm, out_oh_hbm,
                        idx_v, table_v, rows_v, oh_v, sem):
    wid = lax.axis_index("s") * _NC + lax.axis_index("c")
    base = wid * _BPW

    idx_cp = pltpu.async_copy(stages_hbm.at[pl.ds(base, _BPW)], idx_v, sem)
    tab_cp = pltpu.async_copy(
        table_hbm.at[pl.ds(wid * (_NUM_STAGES * _EMBED_DIM),
                           _NUM_STAGES * _EMBED_DIM)],
        table_v, sem,
    )
    idx_cp.wait()
    tab_cp.wait()

    ones = jnp.ones((_L,), jnp.float32)
    zeros = jnp.zeros((_L,), jnp.float32)

    def chunk_body(c, carry):
        s_chunk = idx_v[pl.ds(c * _L, _L)]
        for j in range(_NUM_STAGES):
            oh_v[j, pl.ds(c * _L, _L)] = jnp.where(s_chunk == j, ones, zeros)
        srcs = [s_chunk[k] * _EMBED_DIM for k in range(_L)]
        prev = None
        for k in range(_L + 1):
            cur = []
            for v in range(_VPR):
                if k < _L:
                    cur.append(table_v[pl.ds(srcs[k] + v * _L, _L)])
                if prev is not None:
                    rows_v[c * _L + k - 1, pl.ds(v * _L, _L)] = prev[v]
            prev = cur

        @pl.when(c % _CPQ == _CPQ - 1)
        def _():
            q = c // _CPQ
            pltpu.make_async_copy(
                rows_v.at[pl.ds(q * _RPQ, _RPQ)],
                out_emb_hbm.at[pl.ds(base + q * _RPQ, _RPQ)],
                sem,
            ).start()

        return carry

    lax.fori_loop(0, _CHUNKS, chunk_body, 0)
    oh_copy = pltpu.async_copy(oh_v, out_oh_hbm.at[:, pl.ds(base, _BPW)], sem)
    for q in range(_QUARTERS):
        pltpu.make_async_copy(
            rows_v.at[pl.ds(q * _RPQ, _RPQ)],
            out_emb_hbm.at[pl.ds(base + q * _RPQ, _RPQ)],
            sem,
        ).wait()
    oh_copy.wait()


def kernel(stages, table):
    stages_i32 = stages.reshape(-1).astype(jnp.int32)
    table_rep = jnp.tile(table.reshape(-1), _NW)
    emb, oh_t = _stage_embed_kernel(stages_i32, table_rep)
    return emb, oh_t.T


# speedup vs baseline: 5.0332x; 1.0154x over previous
import functools

import jax
import jax.numpy as jnp
from jax import lax
from jax.experimental import pallas as pl
from jax.experimental.pallas import tpu as pltpu
from jax.experimental.pallas import tpu_sc as plsc

_NUM_STAGES = 10
_EMBED_DIM = 128
_BATCH = 16384
_NC = 2
_NS = 16
_L = 16
_NW = _NC * _NS
_BPW = _BATCH // _NW
_CHUNKS = _BPW // _L
_VPR = _EMBED_DIM // _L
_OH_WORDS = _BPW * _NUM_STAGES
_QUARTERS = 8
_CPQ = _CHUNKS // _QUARTERS
_RPQ = _BPW // _QUARTERS

_mesh = plsc.VectorSubcoreMesh(core_axis_name="c", subcore_axis_name="s")


@functools.partial(
    pl.kernel,
    mesh=_mesh,
    out_type=jax.ShapeDtypeStruct((_BATCH, _EMBED_DIM), jnp.float32),
    scratch_types=[
        pltpu.VMEM((_BPW,), jnp.int32),
        pltpu.VMEM((_NUM_STAGES * _EMBED_DIM,), jnp.float32),
        pltpu.VMEM((_BPW, _EMBED_DIM), jnp.float32),
        pltpu.SemaphoreType.DMA,
    ],
    compiler_params=pltpu.CompilerParams(
        needs_layout_passes=False, skip_device_barrier=True
    ),
)
def _stage_embed_kernel(stages_hbm, table_hbm, out_emb_hbm,
                        idx_v, table_v, rows_v, sem):
    wid = lax.axis_index("s") * _NC + lax.axis_index("c")
    base = wid * _BPW

    idx_cp = pltpu.async_copy(stages_hbm.at[pl.ds(base, _BPW)], idx_v, sem)
    tab_cp = pltpu.async_copy(
        table_hbm.at[pl.ds(wid * (_NUM_STAGES * _EMBED_DIM),
                           _NUM_STAGES * _EMBED_DIM)],
        table_v, sem,
    )
    idx_cp.wait()
    tab_cp.wait()

    def chunk_body(c, carry):
        s_chunk = idx_v[pl.ds(c * _L, _L)]
        srcs = [s_chunk[k] * _EMBED_DIM for k in range(_L)]
        prev = None
        for k in range(_L + 1):
            cur = []
            for v in range(_VPR):
                if k < _L:
                    cur.append(table_v[pl.ds(srcs[k] + v * _L, _L)])
                if prev is not None:
                    rows_v[c * _L + k - 1, pl.ds(v * _L, _L)] = prev[v]
            prev = cur

        @pl.when(c % _CPQ == _CPQ - 1)
        def _():
            q = c // _CPQ
            pltpu.make_async_copy(
                rows_v.at[pl.ds(q * _RPQ, _RPQ)],
                out_emb_hbm.at[pl.ds(base + q * _RPQ, _RPQ)],
                sem,
            ).start()

        return carry

    lax.fori_loop(0, _CHUNKS, chunk_body, 0)
    for q in range(_QUARTERS):
        pltpu.make_async_copy(
            rows_v.at[pl.ds(q * _RPQ, _RPQ)],
            out_emb_hbm.at[pl.ds(base + q * _RPQ, _RPQ)],
            sem,
        ).wait()


def _onehot_body(stages_ref, out_ref):
    s = stages_ref[...]
    stage_ids = lax.broadcasted_iota(jnp.int32, (_NUM_STAGES, _BATCH), 0)
    out_ref[...] = jnp.where(stage_ids == s, 1.0, 0.0).astype(jnp.float32)


_onehot_tc = pl.pallas_call(
    _onehot_body,
    out_shape=jax.ShapeDtypeStruct((_NUM_STAGES, _BATCH), jnp.float32),
)


def kernel(stages, table):
    stages_i32 = stages.reshape(-1).astype(jnp.int32)
    table_rep = jnp.tile(table.reshape(-1), _NW)
    emb = _stage_embed_kernel(stages_i32, table_rep)
    oh_t = _onehot_tc(stages_i32.reshape(1, _BATCH))
    return emb, oh_t.T
